# baseline (speedup 1.0000x reference)
import jax
import jax.numpy as jnp
from jax import lax
from jax.experimental import pallas as pl
from jax.experimental.pallas import tpu as pltpu
from jax.experimental.pallas import tpu_sc as plsc

_N = 10000
_E = 160000
_IN, _H1, _H2, _OUT = 256, 1024, 512, 256

_NC, _NS = 2, 16
_FC = 128
_B = 128
_BPT = 80
_EPAD = _NS * _BPT * _B
_NP = 10112
_RPT = _NP // _NS
_DSTPAD = _NP - 1

_RB = 1000


def _sc_segsum(table_list, src, dst, zeros2d, zeros1d, with_count):
  C = len(table_list)
  S = 2 if with_count else 3
  mesh = plsc.VectorSubcoreMesh(
      core_axis_name="c", subcore_axis_name="s",
      num_cores=_NC, num_subcores=_NS)

  out_type = [jax.ShapeDtypeStruct((_NP, _FC), jnp.float32) for _ in range(C)]
  if with_count:
    out_type.append(jax.ShapeDtypeStruct((_NP,), jnp.float32))

  scratch = (
      [pltpu.VMEM((_B,), jnp.int32) for _ in range(S)]
      + [pltpu.VMEM((_B,), jnp.int32) for _ in range(S)]
      + [pltpu.VMEM((_B, _FC), jnp.float32) for _ in range(S)]
      + [pltpu.VMEM((_B,), jnp.float32)]
      + [pltpu.VMEM_SHARED((_NP, _FC), jnp.float32)]
  )
  if with_count:
    scratch.append(pltpu.VMEM_SHARED((_NP,), jnp.float32))
  scratch.extend([pltpu.SemaphoreType.DMA] * (2 * S))

  def body(*refs):
    tables = refs[:C]
    src_hbm, dst_hbm, zeros2d_hbm = refs[C], refs[C + 1], refs[C + 2]
    i = C + 3
    if with_count:
      zeros1d_hbm = refs[i]
      i += 1
    outs = refs[i:i + C]
    i += C
    if with_count:
      cnt_hbm = refs[i]
      i += 1
    src_v = refs[i:i + S]
    dst_v = refs[i + S:i + 2 * S]
    rows_v = refs[i + 2 * S:i + 3 * S]
    ones_v = refs[i + 3 * S]
    acc = refs[i + 3 * S + 1]
    i += 3 * S + 2
    if with_count:
      cntacc = refs[i]
      i += 1
    semg = refs[i:i + S]
    sems = refs[i + S:i + 2 * S]

    cid = lax.axis_index("c")
    sid = lax.axis_index("s")
    rbase = sid * _RPT
    ebase = sid * _BPT * _B

    if with_count:
      for l in range(_B // 16):
        ones_v[pl.ds(l * 16, 16)] = jnp.ones((16,), jnp.float32)

    for c in range(C):
      @pl.when(cid == (c % _NC))
      def _(c=c):
        pltpu.sync_copy(zeros2d_hbm, acc.at[pl.ds(rbase, _RPT)])
        if with_count and c == 0:
          @pl.when(sid == 0)
          def _():
            pltpu.sync_copy(zeros1d_hbm, cntacc)
        plsc.subcore_barrier()

        def fetch(m, q):
          off = ebase + m * _B
          pltpu.sync_copy(src_hbm.at[pl.ds(off, _B)], src_v[q])
          pltpu.sync_copy(dst_hbm.at[pl.ds(off, _B)], dst_v[q])
          pltpu.async_copy(tables[c].at[src_v[q]], rows_v[q], semg[q])

        def wait_gather(q):
          pltpu.make_async_copy(tables[c].at[src_v[q]], rows_v[q],
                                semg[q]).wait()

        def issue_scatter(q):
          pltpu.async_copy(rows_v[q], acc.at[dst_v[q]], sems[q], add=True)
          if with_count and c == 0:
            pltpu.async_copy(ones_v, cntacc.at[dst_v[q]], sems[q], add=True)

        def wait_scatter(q):
          pltpu.make_async_copy(rows_v[q], acc.at[dst_v[q]], sems[q]).wait()
          if with_count and c == 0:
            pltpu.make_async_copy(ones_v, cntacc.at[dst_v[q]],
                                  sems[q]).wait()

        if S == 2:
          fetch(0, 0)

          def pair(k2, carry):
            for q in range(2):
              m = 2 * k2 + q

              @pl.when(m < _BPT)
              def _(m=m, q=q):
                @pl.when(m + 1 < _BPT)
                def _(m=m, q=q):
                  @pl.when(m >= 1)
                  def _(q=q):
                    wait_scatter(1 - q)
                  fetch(m + 1, 1 - q)
                wait_gather(q)
                issue_scatter(q)
            return carry

          lax.fori_loop(0, (_BPT + 1) // 2, pair, 0)
          wait_scatter(0)
          wait_scatter(1)
        else:
          fetch(0, 0)
          fetch(1, 1)

          def triple(k3, carry):
            for j in range(3):
              m = 3 * k3 + j

              @pl.when(m < _BPT)
              def _(m=m, j=j):
                wait_gather(j)
                issue_scatter(j)

                @pl.when(m >= 1)
                def _(j=j):
                  wait_scatter((j - 1) % 3)

                @pl.when(m + 2 < _BPT)
                def _(m=m, j=j):
                  fetch(m + 2, (j + 2) % 3)
            return carry

          lax.fori_loop(0, (_BPT + 2) // 3, triple, 0)
          wait_scatter((_BPT - 1) % 3)

        plsc.subcore_barrier()
        pltpu.sync_copy(acc.at[pl.ds(rbase, _RPT)],
                        outs[c].at[pl.ds(rbase, _RPT)])
        if with_count and c == 0:
          @pl.when(sid == 0)
          def _():
            pltpu.sync_copy(cntacc, cnt_hbm)
        plsc.subcore_barrier()

    return None

  k = pl.kernel(body, out_type=tuple(out_type), mesh=mesh,
                scratch_types=tuple(scratch))
  args = list(table_list) + [src, dst, zeros2d]
  if with_count:
    args.append(zeros1d)
  return k(*args)


def _tc_layer1(x, agg0, agg1, cnt2, w1lT, b1l2, w1rT, w2lT, b2l2, w2rT):
  def body(x_ref, a0_ref, a1_ref, cnt_ref, w1l_ref, b1l_ref, w1r_ref,
           w2l_ref, b2l_ref, w2r_ref, hl0, hl1, hl2, hl3, hr_ref):
    rcp = 1.0 / jnp.maximum(cnt_ref[...], 1.0)
    mean = jnp.concatenate([a0_ref[...], a1_ref[...]], axis=1) * rcp
    t = (jnp.dot(mean, w1l_ref[...], preferred_element_type=jnp.float32)
         + jnp.dot(x_ref[...], w1r_ref[...], preferred_element_type=jnp.float32)
         + b1l_ref[...])
    h = jnp.maximum(t, 0.0)
    hl = jnp.dot(h, w2l_ref[...], preferred_element_type=jnp.float32)
    hr = (jnp.dot(h, w2r_ref[...], preferred_element_type=jnp.float32)
          + b2l_ref[...])
    hl0[...] = hl[:, 0:128]
    hl1[...] = hl[:, 128:256]
    hl2[...] = hl[:, 256:384]
    hl3[...] = hl[:, 384:512]
    hr_ref[...] = hr

  row = lambda i: (i, 0)
  fixed = lambda i: (0, 0)
  return pl.pallas_call(
      body,
      grid=(_N // _RB,),
      in_specs=[
          pl.BlockSpec((_RB, _IN), row),
          pl.BlockSpec((_RB, _FC), row),
          pl.BlockSpec((_RB, _FC), row),
          pl.BlockSpec((_RB, 1), row),
          pl.BlockSpec((_IN, _H1), fixed),
          pl.BlockSpec((1, _H1), fixed),
          pl.BlockSpec((_IN, _H1), fixed),
          pl.BlockSpec((_H1, _H2), fixed),
          pl.BlockSpec((1, _H2), fixed),
          pl.BlockSpec((_H1, _H2), fixed),
      ],
      out_specs=[
          pl.BlockSpec((_RB, _FC), row),
          pl.BlockSpec((_RB, _FC), row),
          pl.BlockSpec((_RB, _FC), row),
          pl.BlockSpec((_RB, _FC), row),
          pl.BlockSpec((_RB, _H2), row),
      ],
      out_shape=[
          jax.ShapeDtypeStruct((_N, _FC), jnp.float32),
          jax.ShapeDtypeStruct((_N, _FC), jnp.float32),
          jax.ShapeDtypeStruct((_N, _FC), jnp.float32),
          jax.ShapeDtypeStruct((_N, _FC), jnp.float32),
          jax.ShapeDtypeStruct((_N, _H2), jnp.float32),
      ],
  )(x, agg0, agg1, cnt2, w1lT, b1l2, w1rT, w2lT, b2l2, w2rT)


def _tc_layer2(a0, a1, a2, a3, cnt2, hr, wmuT, bmu2, wlvT, blv2):
  def body(a0_ref, a1_ref, a2_ref, a3_ref, cnt_ref, hr_ref,
           wmu_ref, bmu_ref, wlv_ref, blv_ref, mu_ref, lv_ref):
    rcp = 1.0 / jnp.maximum(cnt_ref[...], 1.0)
    agg = jnp.concatenate(
        [a0_ref[...], a1_ref[...], a2_ref[...], a3_ref[...]], axis=1)
    h2 = agg * rcp + hr_ref[...]
    mu_ref[...] = (jnp.dot(h2, wmu_ref[...],
                           preferred_element_type=jnp.float32) + bmu_ref[...])
    lv_ref[...] = (jnp.dot(h2, wlv_ref[...],
                           preferred_element_type=jnp.float32) + blv_ref[...])

  row = lambda i: (i, 0)
  fixed = lambda i: (0, 0)
  return pl.pallas_call(
      body,
      grid=(_N // _RB,),
      in_specs=[
          pl.BlockSpec((_RB, _FC), row),
          pl.BlockSpec((_RB, _FC), row),
          pl.BlockSpec((_RB, _FC), row),
          pl.BlockSpec((_RB, _FC), row),
          pl.BlockSpec((_RB, 1), row),
          pl.BlockSpec((_RB, _H2), row),
          pl.BlockSpec((_H2, _OUT), fixed),
          pl.BlockSpec((1, _OUT), fixed),
          pl.BlockSpec((_H2, _OUT), fixed),
          pl.BlockSpec((1, _OUT), fixed),
      ],
      out_specs=[
          pl.BlockSpec((_RB, _OUT), row),
          pl.BlockSpec((_RB, _OUT), row),
      ],
      out_shape=[
          jax.ShapeDtypeStruct((_N, _OUT), jnp.float32),
          jax.ShapeDtypeStruct((_N, _OUT), jnp.float32),
      ],
  )(a0, a1, a2, a3, cnt2, hr, wmuT, bmu2, wlvT, blv2)


def kernel(x, edge_index, W1l, b1l, W1r, W2l, b2l, W2r, Wmu, bmu, Wlv, blv):
  f32 = jnp.float32
  src = jnp.concatenate(
      [edge_index[0], jnp.zeros((_EPAD - _E,), jnp.int32)])
  dst = jnp.concatenate(
      [edge_index[1], jnp.full((_EPAD - _E,), _DSTPAD, jnp.int32)])
  x0 = x[:, :_FC]
  x1 = x[:, _FC:]
  zeros2d = jnp.zeros((_RPT, _FC), f32)
  zeros1d = jnp.zeros((_NP,), f32)

  agg10, agg11, cnt = _sc_segsum([x0, x1], src, dst, zeros2d, zeros1d, True)
  cnt2 = cnt.reshape(_NP, 1)

  hl0, hl1, hl2, hl3, hr = _tc_layer1(
      x, agg10, agg11, cnt2, W1l.T, b1l.reshape(1, -1), W1r.T,
      W2l.T, b2l.reshape(1, -1), W2r.T)

  a20, a21, a22, a23 = _sc_segsum(
      [hl0, hl1, hl2, hl3], src, dst, zeros2d, zeros1d, False)

  mu, lv = _tc_layer2(
      a20, a21, a22, a23, cnt2, hr, Wmu.T, bmu.reshape(1, -1),
      Wlv.T, blv.reshape(1, -1))
  return (mu, lv)


# speedup vs baseline: 2.0306x; 2.0306x over previous
import jax
import jax.numpy as jnp
from jax import lax
from jax.experimental import pallas as pl
from jax.experimental.pallas import tpu as pltpu
from jax.experimental.pallas import tpu_sc as plsc

_N = 10000
_E = 160000
_IN, _H1, _H2, _OUT = 256, 1024, 512, 256

_NC, _NS = 2, 16
_FC = 128
_B = 128
_BPT = 80
_EPAD = _NS * _BPT * _B
_NP = 10112
_RPT = _NP // _NS
_DSTPAD = _NP - 1

_RB = 1000


def _sc_segsum(table_list, src, dst, zeros2d, zeros1d, with_count):
  C = len(table_list)
  S = 2 if with_count else 3
  mesh = plsc.VectorSubcoreMesh(
      core_axis_name="c", subcore_axis_name="s",
      num_cores=_NC, num_subcores=_NS)

  out_type = [jax.ShapeDtypeStruct((_NP, _FC), jnp.float32) for _ in range(C)]
  if with_count:
    out_type.append(jax.ShapeDtypeStruct((_NP,), jnp.float32))

  scratch = (
      [pltpu.VMEM((_B,), jnp.int32) for _ in range(S)]
      + [pltpu.VMEM((_B,), jnp.int32) for _ in range(S)]
      + [pltpu.VMEM((_B, _FC), jnp.float32) for _ in range(S)]
      + [pltpu.VMEM((_B,), jnp.float32)]
      + [pltpu.VMEM_SHARED((_NP, _FC), jnp.float32)]
  )
  if with_count:
    scratch.append(pltpu.VMEM_SHARED((_NP,), jnp.float32))
  scratch.extend([pltpu.SemaphoreType.DMA] * (2 * S))

  def body(*refs):
    tables = refs[:C]
    src_hbm, dst_hbm, zeros2d_hbm = refs[C], refs[C + 1], refs[C + 2]
    i = C + 3
    if with_count:
      zeros1d_hbm = refs[i]
      i += 1
    outs = refs[i:i + C]
    i += C
    if with_count:
      cnt_hbm = refs[i]
      i += 1
    src_v = refs[i:i + S]
    dst_v = refs[i + S:i + 2 * S]
    rows_v = refs[i + 2 * S:i + 3 * S]
    ones_v = refs[i + 3 * S]
    acc = refs[i + 3 * S + 1]
    i += 3 * S + 2
    if with_count:
      cntacc = refs[i]
      i += 1
    semg = refs[i:i + S]
    sems = refs[i + S:i + 2 * S]

    cid = lax.axis_index("c")
    sid = lax.axis_index("s")
    rbase = sid * _RPT
    ebase = sid * _BPT * _B

    if with_count:
      for l in range(_B // 16):
        ones_v[pl.ds(l * 16, 16)] = jnp.ones((16,), jnp.float32)

    for c in range(C):
      @pl.when(cid == (c % _NC))
      def _(c=c):
        pltpu.sync_copy(zeros2d_hbm, acc.at[pl.ds(rbase, _RPT)])
        if with_count and c == 0:
          @pl.when(sid == 0)
          def _():
            pltpu.sync_copy(zeros1d_hbm, cntacc)
        plsc.subcore_barrier()

        def fetch(m, q):
          off = ebase + m * _B
          pltpu.sync_copy(src_hbm.at[pl.ds(off, _B)], src_v[q])
          pltpu.sync_copy(dst_hbm.at[pl.ds(off, _B)], dst_v[q])
          pltpu.async_copy(tables[c].at[src_v[q]], rows_v[q], semg[q])

        def wait_gather(q):
          pltpu.make_async_copy(tables[c].at[src_v[q]], rows_v[q],
                                semg[q]).wait()

        def issue_scatter(q):
          pltpu.async_copy(rows_v[q], acc.at[dst_v[q]], sems[q], add=True)
          if with_count and c == 0:
            pltpu.async_copy(ones_v, cntacc.at[dst_v[q]], sems[q], add=True)

        def wait_scatter(q):
          pltpu.make_async_copy(rows_v[q], acc.at[dst_v[q]], sems[q]).wait()
          if with_count and c == 0:
            pltpu.make_async_copy(ones_v, cntacc.at[dst_v[q]],
                                  sems[q]).wait()

        if S == 2:
          fetch(0, 0)

          def pair(k2, carry):
            for q in range(2):
              m = 2 * k2 + q

              @pl.when(m < _BPT)
              def _(m=m, q=q):
                @pl.when(m + 1 < _BPT)
                def _(m=m, q=q):
                  @pl.when(m >= 1)
                  def _(q=q):
                    wait_scatter(1 - q)
                  fetch(m + 1, 1 - q)
                wait_gather(q)
                issue_scatter(q)
            return carry

          lax.fori_loop(0, (_BPT + 1) // 2, pair, 0)
          wait_scatter(0)
          wait_scatter(1)
        else:
          fetch(0, 0)
          fetch(1, 1)

          def triple(k3, carry):
            for j in range(3):
              m = 3 * k3 + j

              @pl.when(m < _BPT)
              def _(m=m, j=j):
                wait_gather(j)
                issue_scatter(j)

                @pl.when(m >= 1)
                def _(j=j):
                  wait_scatter((j - 1) % 3)

                @pl.when(m + 2 < _BPT)
                def _(m=m, j=j):
                  fetch(m + 2, (j + 2) % 3)
            return carry

          lax.fori_loop(0, (_BPT + 2) // 3, triple, 0)
          wait_scatter((_BPT - 1) % 3)

        plsc.subcore_barrier()
        pltpu.sync_copy(acc.at[pl.ds(rbase, _RPT)],
                        outs[c].at[pl.ds(rbase, _RPT)])
        if with_count and c == 0:
          @pl.when(sid == 0)
          def _():
            pltpu.sync_copy(cntacc, cnt_hbm)
        plsc.subcore_barrier()

    return None

  k = pl.kernel(body, out_type=tuple(out_type), mesh=mesh,
                scratch_types=tuple(scratch))
  args = list(table_list) + [src, dst, zeros2d]
  if with_count:
    args.append(zeros1d)
  return k(*args)


def _tc_layer1(x, agg0, agg1, cnt2, w1lT, b1l2, w1rT, w2lT, b2l2, w2rT):
  def body(x_ref, a0_ref, a1_ref, cnt_ref, w1l_ref, b1l_ref, w1r_ref,
           w2l_ref, b2l_ref, w2r_ref, hl0, hl1, hl2, hl3, hr_ref):
    rcp = 1.0 / jnp.maximum(cnt_ref[...], 1.0)
    mean = jnp.concatenate([a0_ref[...], a1_ref[...]], axis=1) * rcp
    t = (jnp.dot(mean, w1l_ref[...], preferred_element_type=jnp.float32)
         + jnp.dot(x_ref[...], w1r_ref[...], preferred_element_type=jnp.float32)
         + b1l_ref[...])
    h = jnp.maximum(t, 0.0)
    hl = jnp.dot(h, w2l_ref[...], preferred_element_type=jnp.float32)
    hr = (jnp.dot(h, w2r_ref[...], preferred_element_type=jnp.float32)
          + b2l_ref[...])
    hl0[...] = hl[:, 0:128]
    hl1[...] = hl[:, 128:256]
    hl2[...] = hl[:, 256:384]
    hl3[...] = hl[:, 384:512]
    hr_ref[...] = hr

  row = lambda i: (i, 0)
  fixed = lambda i: (0, 0)
  return pl.pallas_call(
      body,
      grid=(_N // _RB,),
      in_specs=[
          pl.BlockSpec((_RB, _IN), row),
          pl.BlockSpec((_RB, _FC), row),
          pl.BlockSpec((_RB, _FC), row),
          pl.BlockSpec((_RB, 1), row),
          pl.BlockSpec((_IN, _H1), fixed),
          pl.BlockSpec((1, _H1), fixed),
          pl.BlockSpec((_IN, _H1), fixed),
          pl.BlockSpec((_H1, _H2), fixed),
          pl.BlockSpec((1, _H2), fixed),
          pl.BlockSpec((_H1, _H2), fixed),
      ],
      out_specs=[
          pl.BlockSpec((_RB, _FC), row),
          pl.BlockSpec((_RB, _FC), row),
          pl.BlockSpec((_RB, _FC), row),
          pl.BlockSpec((_RB, _FC), row),
          pl.BlockSpec((_RB, _H2), row),
      ],
      out_shape=[
          jax.ShapeDtypeStruct((_N, _FC), jnp.float32),
          jax.ShapeDtypeStruct((_N, _FC), jnp.float32),
          jax.ShapeDtypeStruct((_N, _FC), jnp.float32),
          jax.ShapeDtypeStruct((_N, _FC), jnp.float32),
          jax.ShapeDtypeStruct((_N, _H2), jnp.float32),
      ],
  )(x, agg0, agg1, cnt2, w1lT, b1l2, w1rT, w2lT, b2l2, w2rT)


def _tc_layer2(a0, a1, a2, a3, cnt2, hr, wmuT, bmu2, wlvT, blv2):
  def body(a0_ref, a1_ref, a2_ref, a3_ref, cnt_ref, hr_ref,
           wmu_ref, bmu_ref, wlv_ref, blv_ref, mu_ref, lv_ref):
    rcp = 1.0 / jnp.maximum(cnt_ref[...], 1.0)
    agg = jnp.concatenate(
        [a0_ref[...], a1_ref[...], a2_ref[...], a3_ref[...]], axis=1)
    h2 = agg * rcp + hr_ref[...]
    mu_ref[...] = (jnp.dot(h2, wmu_ref[...],
                           preferred_element_type=jnp.float32) + bmu_ref[...])
    lv_ref[...] = (jnp.dot(h2, wlv_ref[...],
                           preferred_element_type=jnp.float32) + blv_ref[...])

  row = lambda i: (i, 0)
  fixed = lambda i: (0, 0)
  return pl.pallas_call(
      body,
      grid=(_N // _RB,),
      in_specs=[
          pl.BlockSpec((_RB, _FC), row),
          pl.BlockSpec((_RB, _FC), row),
          pl.BlockSpec((_RB, _FC), row),
          pl.BlockSpec((_RB, _FC), row),
          pl.BlockSpec((_RB, 1), row),
          pl.BlockSpec((_RB, _H2), row),
          pl.BlockSpec((_H2, _OUT), fixed),
          pl.BlockSpec((1, _OUT), fixed),
          pl.BlockSpec((_H2, _OUT), fixed),
          pl.BlockSpec((1, _OUT), fixed),
      ],
      out_specs=[
          pl.BlockSpec((_RB, _OUT), row),
          pl.BlockSpec((_RB, _OUT), row),
      ],
      out_shape=[
          jax.ShapeDtypeStruct((_N, _OUT), jnp.float32),
          jax.ShapeDtypeStruct((_N, _OUT), jnp.float32),
      ],
  )(a0, a1, a2, a3, cnt2, hr, wmuT, bmu2, wlvT, blv2)


def kernel(x, edge_index, W1l, b1l, W1r, W2l, b2l, W2r, Wmu, bmu, Wlv, blv):
  f32 = jnp.float32
  npad = _EPAD - _E
  src = jnp.concatenate(
      [edge_index[0], (jnp.arange(npad, dtype=jnp.int32) * 16) % _N])
  dst = jnp.concatenate(
      [edge_index[1],
       _N + (jnp.arange(npad, dtype=jnp.int32) % (_NP - _N))])
  x0 = x[:, :_FC]
  x1 = x[:, _FC:]
  zeros2d = jnp.zeros((_RPT, _FC), f32)
  zeros1d = jnp.zeros((_NP,), f32)

  agg10, agg11, cnt = _sc_segsum([x0, x1], src, dst, zeros2d, zeros1d, True)
  cnt2 = cnt.reshape(_NP, 1)

  hl0, hl1, hl2, hl3, hr = _tc_layer1(
      x, agg10, agg11, cnt2, W1l.T, b1l.reshape(1, -1), W1r.T,
      W2l.T, b2l.reshape(1, -1), W2r.T)

  a20, a21, a22, a23 = _sc_segsum(
      [hl0, hl1, hl2, hl3], src, dst, zeros2d, zeros1d, False)

  mu, lv = _tc_layer2(
      a20, a21, a22, a23, cnt2, hr, Wmu.T, bmu.reshape(1, -1),
      Wlv.T, blv.reshape(1, -1))
  return (mu, lv)


# speedup vs baseline: 2.2821x; 1.1239x over previous
import jax
import jax.numpy as jnp
from jax import lax
from jax.experimental import pallas as pl
from jax.experimental.pallas import tpu as pltpu
from jax.experimental.pallas import tpu_sc as plsc

_N = 10000
_E = 160000
_IN, _H1, _H2, _OUT = 256, 1024, 512, 256

_NC, _NS = 2, 16
_FC = 128
_NP = 10112
_RPT = _NP // _NS

_RB = 1000
_dn = (((1,), (1,)), ((), ()))


def _sc_segsum(table_list, edges, zeros2d, zeros1d, with_count, B, BPT, S):
  C = len(table_list)
  D = 2
  L = S - D
  mesh = plsc.VectorSubcoreMesh(
      core_axis_name="c", subcore_axis_name="s",
      num_cores=_NC, num_subcores=_NS)

  out_type = [jax.ShapeDtypeStruct((_NP, _FC), jnp.float32) for _ in range(C)]
  if with_count:
    out_type.append(jax.ShapeDtypeStruct((_NP,), jnp.float32))
    out_type.append(jax.ShapeDtypeStruct((_NP,), jnp.float32))

  scratch = (
      [pltpu.VMEM((2, B), jnp.int32) for _ in range(S)]
      + [pltpu.VMEM((B, _FC), jnp.float32) for _ in range(S)]
      + [pltpu.VMEM((B,), jnp.float32)]
      + [pltpu.VMEM_SHARED((_NP, _FC), jnp.float32)]
  )
  if with_count:
    scratch.append(pltpu.VMEM_SHARED((_NP,), jnp.float32))
  scratch.extend([pltpu.SemaphoreType.DMA] * (2 * S))

  def body(*refs):
    tables = refs[:C]
    edges_hbm, zeros2d_hbm = refs[C], refs[C + 1]
    i = C + 2
    if with_count:
      zeros1d_hbm = refs[i]
      i += 1
    outs = refs[i:i + C]
    i += C
    if with_count:
      cnt_hbm = (refs[i], refs[i + 1])
      i += 2
    ebuf = refs[i:i + S]
    rows_v = refs[i + S:i + 2 * S]
    ones_v = refs[i + 2 * S]
    acc = refs[i + 2 * S + 1]
    i += 2 * S + 2
    if with_count:
      cntacc = refs[i]
      i += 1
    semg = refs[i:i + S]
    sems = refs[i + S:i + 2 * S]

    cid = lax.axis_index("c")
    sid = lax.axis_index("s")
    rbase = sid * _RPT

    if with_count:
      for l in range(B // 16):
        ones_v[pl.ds(l * 16, 16)] = jnp.ones((16,), jnp.float32)

    for c in range(C):
      @pl.when(cid == (c % _NC))
      def _(c=c):
        pltpu.sync_copy(zeros2d_hbm, acc.at[pl.ds(rbase, _RPT)])
        if with_count and c < _NC:
          @pl.when(sid == 0)
          def _():
            pltpu.sync_copy(zeros1d_hbm, cntacc)
        plsc.subcore_barrier()

        def fetch(m, q):
          pltpu.sync_copy(edges_hbm.at[sid, m], ebuf[q])
          pltpu.async_copy(tables[c].at[ebuf[q].at[0]], rows_v[q], semg[q])

        def wait_gather(q):
          pltpu.make_async_copy(tables[c].at[ebuf[q].at[0]], rows_v[q],
                                semg[q]).wait()

        def issue_scatter(m, q):
          pltpu.async_copy(rows_v[q], acc.at[ebuf[q].at[1]], sems[q],
                           add=True)
          if with_count and c < _NC:
            @pl.when(m % 2 == cid)
            def _():
              pltpu.async_copy(ones_v, cntacc.at[ebuf[q].at[1]], sems[q],
                               add=True)

        def wait_scatter(m, q):
          pltpu.make_async_copy(rows_v[q], acc.at[ebuf[q].at[1]],
                                sems[q]).wait()
          if with_count and c < _NC:
            @pl.when(m % 2 == cid)
            def _():
              pltpu.make_async_copy(ones_v, cntacc.at[ebuf[q].at[1]],
                                    sems[q]).wait()

        for d in range(D):
          fetch(d, d)

        def group(kS, carry):
          for j in range(S):
            m = S * kS + j

            @pl.when(m < BPT)
            def _(m=m, j=j):
              wait_gather(j)
              issue_scatter(m, j)

              @pl.when(m >= L)
              def _(m=m, j=j):
                wait_scatter(m - L, (j - L) % S)

              @pl.when(m + D < BPT)
              def _(m=m, j=j):
                fetch(m + D, (j + D) % S)
          return carry

        lax.fori_loop(0, (BPT + S - 1) // S, group, 0)
        for t in range(L):
          wait_scatter(BPT - L + t, (BPT - L + t) % S)

        plsc.subcore_barrier()
        pltpu.sync_copy(acc.at[pl.ds(rbase, _RPT)],
                        outs[c].at[pl.ds(rbase, _RPT)])
        if with_count and c < _NC:
          @pl.when(sid == 0)
          def _():
            pltpu.sync_copy(cntacc, cnt_hbm[c])
        plsc.subcore_barrier()

    return None

  k = pl.kernel(body, out_type=tuple(out_type), mesh=mesh,
                scratch_types=tuple(scratch))
  args = list(table_list) + [edges, zeros2d]
  if with_count:
    args.append(zeros1d)
  return k(*args)


def _tc_layer1(x, agg0, agg1, ca, cb, w1lT, b1l2, w1rT, w2lT, b2l2, w2rT):
  def body(x_ref, a0_ref, a1_ref, ca_ref, cb_ref, w1l_ref, b1l_ref,
           w1r_ref, w2l_ref, b2l_ref, w2r_ref, hl0, hl1, hl2, hl3, hr_ref):
    rcp = 1.0 / jnp.maximum(ca_ref[...] + cb_ref[...], 1.0)
    mean = jnp.concatenate([a0_ref[...], a1_ref[...]], axis=1) * rcp
    t = (lax.dot_general(mean, w1l_ref[...], _dn,
                         preferred_element_type=jnp.float32)
         + lax.dot_general(x_ref[...], w1r_ref[...], _dn,
                           preferred_element_type=jnp.float32)
         + b1l_ref[...])
    h = jnp.maximum(t, 0.0)
    hl = lax.dot_general(h, w2l_ref[...], _dn,
                         preferred_element_type=jnp.float32)
    hr = (lax.dot_general(h, w2r_ref[...], _dn,
                          preferred_element_type=jnp.float32)
          + b2l_ref[...])
    hl0[...] = hl[:, 0:128]
    hl1[...] = hl[:, 128:256]
    hl2[...] = hl[:, 256:384]
    hl3[...] = hl[:, 384:512]
    hr_ref[...] = hr

  row = lambda i: (i, 0)
  fixed = lambda i: (0, 0)
  return pl.pallas_call(
      body,
      grid=(_N // _RB,),
      in_specs=[
          pl.BlockSpec((_RB, _IN), row),
          pl.BlockSpec((_RB, _FC), row),
          pl.BlockSpec((_RB, _FC), row),
          pl.BlockSpec((_RB, 1), row),
          pl.BlockSpec((_RB, 1), row),
          pl.BlockSpec((_H1, _IN), fixed),
          pl.BlockSpec((1, _H1), fixed),
          pl.BlockSpec((_H1, _IN), fixed),
          pl.BlockSpec((_H2, _H1), fixed),
          pl.BlockSpec((1, _H2), fixed),
          pl.BlockSpec((_H2, _H1), fixed),
      ],
      out_specs=[
          pl.BlockSpec((_RB, _FC), row),
          pl.BlockSpec((_RB, _FC), row),
          pl.BlockSpec((_RB, _FC), row),
          pl.BlockSpec((_RB, _FC), row),
          pl.BlockSpec((_RB, _H2), row),
      ],
      out_shape=[
          jax.ShapeDtypeStruct((_N, _FC), jnp.float32),
          jax.ShapeDtypeStruct((_N, _FC), jnp.float32),
          jax.ShapeDtypeStruct((_N, _FC), jnp.float32),
          jax.ShapeDtypeStruct((_N, _FC), jnp.float32),
          jax.ShapeDtypeStruct((_N, _H2), jnp.float32),
      ],
  )(x, agg0, agg1, ca, cb, w1lT, b1l2, w1rT, w2lT, b2l2, w2rT)


def _tc_layer2(a0, a1, a2, a3, ca, cb, hr, wmuT, bmu2, wlvT, blv2):
  def body(a0_ref, a1_ref, a2_ref, a3_ref, ca_ref, cb_ref, hr_ref,
           wmu_ref, bmu_ref, wlv_ref, blv_ref, mu_ref, lv_ref):
    rcp = 1.0 / jnp.maximum(ca_ref[...] + cb_ref[...], 1.0)
    agg = jnp.concatenate(
        [a0_ref[...], a1_ref[...], a2_ref[...], a3_ref[...]], axis=1)
    h2 = agg * rcp + hr_ref[...]
    mu_ref[...] = (lax.dot_general(h2, wmu_ref[...], _dn,
                                   preferred_element_type=jnp.float32)
                   + bmu_ref[...])
    lv_ref[...] = (lax.dot_general(h2, wlv_ref[...], _dn,
                                   preferred_element_type=jnp.float32)
                   + blv_ref[...])

  row = lambda i: (i, 0)
  fixed = lambda i: (0, 0)
  return pl.pallas_call(
      body,
      grid=(_N // _RB,),
      in_specs=[
          pl.BlockSpec((_RB, _FC), row),
          pl.BlockSpec((_RB, _FC), row),
          pl.BlockSpec((_RB, _FC), row),
          pl.BlockSpec((_RB, _FC), row),
          pl.BlockSpec((_RB, 1), row),
          pl.BlockSpec((_RB, 1), row),
          pl.BlockSpec((_RB, _H2), row),
          pl.BlockSpec((_OUT, _H2), fixed),
          pl.BlockSpec((1, _OUT), fixed),
          pl.BlockSpec((_OUT, _H2), fixed),
          pl.BlockSpec((1, _OUT), fixed),
      ],
      out_specs=[
          pl.BlockSpec((_RB, _OUT), row),
          pl.BlockSpec((_RB, _OUT), row),
      ],
      out_shape=[
          jax.ShapeDtypeStruct((_N, _OUT), jnp.float32),
          jax.ShapeDtypeStruct((_N, _OUT), jnp.float32),
      ],
  )(a0, a1, a2, a3, ca, cb, hr, wmuT, bmu2, wlvT, blv2)


def _pad_edges(e, B, BPT):
  npad = _NS * BPT * B - _E
  src = jnp.concatenate(
      [e[0], (jnp.arange(npad, dtype=jnp.int32) * 16) % _N])
  dst = jnp.concatenate(
      [e[1], _N + (jnp.arange(npad, dtype=jnp.int32) % (_NP - _N))])
  return jnp.stack([src.reshape(_NS, BPT, B), dst.reshape(_NS, BPT, B)],
                   axis=2)


def kernel(x, edge_index, W1l, b1l, W1r, W2l, b2l, W2r, Wmu, bmu, Wlv, blv):
  f32 = jnp.float32
  edges1 = _pad_edges(edge_index, 112, 90)
  edges2 = _pad_edges(edge_index, 128, 80)
  x0 = x[:, :_FC]
  x1 = x[:, _FC:]
  zeros2d = jnp.zeros((_RPT, _FC), f32)
  zeros1d = jnp.zeros((_NP,), f32)

  agg10, agg11, cnt0, cnt1 = _sc_segsum(
      [x0, x1], edges1, zeros2d, zeros1d, True, 112, 90, 3)
  ca = cnt0.reshape(_NP, 1)
  cb = cnt1.reshape(_NP, 1)

  hl0, hl1, hl2, hl3, hr = _tc_layer1(
      x, agg10, agg11, ca, cb, W1l, b1l.reshape(1, -1), W1r,
      W2l, b2l.reshape(1, -1), W2r)

  a20, a21, a22, a23 = _sc_segsum(
      [hl0, hl1, hl2, hl3], edges2, zeros2d, zeros1d, False, 128, 80, 3)

  mu, lv = _tc_layer2(
      a20, a21, a22, a23, ca, cb, hr, Wmu, bmu.reshape(1, -1),
      Wlv, blv.reshape(1, -1))
  return (mu, lv)
